# initial kernel scaffold (unmeasured)
import functools

import jax
import jax.numpy as jnp
from jax import lax
from jax.experimental import pallas as pl
from jax.experimental.pallas import tpu as pltpu

N_DEV = 4
SQ = 2048
SKV = 2048
HQ = 8
DH = 128
DM = 1024
BLK = 64
QC = 512
N_CHUNK = SQ // QC
SCALE = 0.08838834764831843


def kernel(x, Wq, K_ext, V_ext, Wo):
    my = lax.axis_index("i")
    xb = x[0].astype(jnp.bfloat16)
    wq = Wq.astype(jnp.bfloat16)
    wo = Wo.astype(jnp.bfloat16)
    kb = lax.dynamic_index_in_dim(K_ext, my, 0, keepdims=False)
    vb = lax.dynamic_index_in_dim(V_ext, my, 0, keepdims=False)
    kb = jnp.transpose(kb, (1, 0, 2)).astype(jnp.bfloat16)
    vb = jnp.transpose(vb, (1, 0, 2)).astype(jnp.bfloat16)

    def body(x_ref, k_hbm, v_hbm, wq_ref, wo_ref, out_ref,
             comm_ref, kg_ref, vg_ref, q_ref, ctx_ref,
             send_sems, recv_sems, kv_sems):
        my_pos = lax.axis_index("i")
        left = lax.rem(my_pos + N_DEV - 1, N_DEV)
        right = lax.rem(my_pos + 1, N_DEV)

        barrier = pltpu.get_barrier_semaphore()
        for nbr in (left, right):
            pl.semaphore_signal(barrier, inc=1, device_id=(nbr,),
                                device_id_type=pl.DeviceIdType.MESH)
        pl.semaphore_wait(barrier, 2)

        comm_ref[0, 0, :, :] = wq_ref[:, :]
        comm_ref[0, 1, :, :] = wo_ref[:, :]
        out_ref[...] = jnp.zeros_like(out_ref)

        def hop(h, carry):
            g = lax.rem(my_pos - h + N_DEV, N_DEV)
            nxt = jnp.minimum(h + 1, N_DEV - 1)

            rdma = pltpu.make_async_remote_copy(
                src_ref=comm_ref.at[h],
                dst_ref=comm_ref.at[nxt],
                send_sem=send_sems.at[h],
                recv_sem=recv_sems.at[nxt],
                device_id=(right,),
                device_id_type=pl.DeviceIdType.MESH,
            )

            @pl.when(h < N_DEV - 1)
            def _():
                rdma.start()

            kcp = pltpu.make_async_copy(
                k_hbm.at[pl.ds(g * HQ, HQ)], kg_ref, kv_sems.at[0])
            vcp = pltpu.make_async_copy(
                v_hbm.at[pl.ds(g * HQ, HQ)], vg_ref, kv_sems.at[1])
            kcp.start()
            vcp.start()
            kcp.wait()
            vcp.wait()

            wq_g = comm_ref[h, 0]
            q_ref[...] = jnp.dot(
                x_ref[...], wq_g,
                preferred_element_type=jnp.float32).astype(jnp.bfloat16)

            for hd in range(HQ):
                kh = kg_ref[hd]
                vh = vg_ref[hd]
                for c in range(N_CHUNK):
                    klen = QC * (c + 1)
                    qc = q_ref[c * QC:(c + 1) * QC, hd * DH:(hd + 1) * DH]
                    s = lax.dot_general(
                        qc, kh[:klen],
                        (((1,), (1,)), ((), ())),
                        preferred_element_type=jnp.float32) * SCALE
                    qb_i = (lax.broadcasted_iota(jnp.int32, (QC, klen), 0)
                            + c * QC) // BLK
                    kb_i = lax.broadcasted_iota(jnp.int32, (QC, klen), 1) // BLK
                    s = jnp.where(kb_i <= qb_i, s, -1e9)
                    m = jnp.max(s, axis=1, keepdims=True)
                    p = jnp.exp(s - m)
                    p = p / jnp.sum(p, axis=1, keepdims=True)
                    ctx_c = jnp.dot(p.astype(jnp.bfloat16), vh[:klen],
                                    preferred_element_type=jnp.float32)
                    ctx_ref[c * QC:(c + 1) * QC,
                            hd * DH:(hd + 1) * DH] = ctx_c.astype(jnp.bfloat16)

            wo_g = comm_ref[h, 1]
            out_ref[...] += jnp.dot(ctx_ref[...], wo_g,
                                    preferred_element_type=jnp.float32)

            @pl.when(h < N_DEV - 1)
            def _():
                rdma.wait()

            return carry

        lax.fori_loop(0, N_DEV, hop, 0)

        @functools.partial(pl.run_scoped,
                           sem2=pltpu.SemaphoreType.REGULAR)
        def _(sem2):
            for nbr in (left, right):
                pl.semaphore_signal(sem2, inc=1, device_id=(nbr,),
                                    device_id_type=pl.DeviceIdType.MESH)
            pl.semaphore_wait(sem2, 2)

    out = pl.pallas_call(
        body,
        out_shape=jax.ShapeDtypeStruct((SQ, DM), jnp.float32),
        in_specs=[
            pl.BlockSpec(memory_space=pltpu.VMEM),
            pl.BlockSpec(memory_space=pltpu.ANY),
            pl.BlockSpec(memory_space=pltpu.ANY),
            pl.BlockSpec(memory_space=pltpu.VMEM),
            pl.BlockSpec(memory_space=pltpu.VMEM),
        ],
        out_specs=pl.BlockSpec(memory_space=pltpu.VMEM),
        scratch_shapes=[
            pltpu.VMEM((N_DEV, 2, DM, DM), jnp.bfloat16),
            pltpu.VMEM((HQ, SKV, DH), jnp.bfloat16),
            pltpu.VMEM((HQ, SKV, DH), jnp.bfloat16),
            pltpu.VMEM((SQ, DM), jnp.bfloat16),
            pltpu.VMEM((SQ, DM), jnp.bfloat16),
            pltpu.SemaphoreType.DMA((N_DEV,)),
            pltpu.SemaphoreType.DMA((N_DEV,)),
            pltpu.SemaphoreType.DMA((2,)),
        ],
        compiler_params=pltpu.CompilerParams(collective_id=0),
    )(xb, kb, vb, wq, wo)
    return out[None]


# baseline (device time: 352593 ns/iter reference)
import functools

import jax
import jax.numpy as jnp
from jax import lax
from jax.experimental import pallas as pl
from jax.experimental.pallas import tpu as pltpu

N_DEV = 4
SQ = 2048
SKV = 2048
HQ = 8
DH = 128
DM = 1024
BLK = 64
QC = 512
N_CHUNK = SQ // QC
SCALE = 0.08838834764831843


def kernel(x, Wq, K_ext, V_ext, Wo):
    my = lax.axis_index("i")
    xb = x[0].astype(jnp.bfloat16)
    wq = Wq.astype(jnp.bfloat16)
    wo = Wo.astype(jnp.bfloat16)
    kb = lax.dynamic_index_in_dim(K_ext, my, 0, keepdims=False)
    vb = lax.dynamic_index_in_dim(V_ext, my, 0, keepdims=False)
    kb = jnp.transpose(kb, (1, 0, 2)).astype(jnp.bfloat16)
    vb = jnp.transpose(vb, (1, 0, 2)).astype(jnp.bfloat16)

    def body(x_ref, k_hbm, v_hbm, wq_ref, wo_ref, out_ref,
             comm_ref, kg_ref, vg_ref,
             send_sems, recv_sems, kv_sems):
        my_pos = lax.axis_index("i")
        left = lax.rem(my_pos + N_DEV - 1, N_DEV)
        right = lax.rem(my_pos + 1, N_DEV)

        barrier = pltpu.get_barrier_semaphore()
        for nbr in (left, right):
            pl.semaphore_signal(barrier, inc=1, device_id=(nbr,),
                                device_id_type=pl.DeviceIdType.MESH)
        pl.semaphore_wait(barrier, 2)

        comm_ref[0, 0, :, :] = wq_ref[:, :]
        comm_ref[0, 1, :, :] = wo_ref[:, :]
        out_ref[...] = jnp.zeros_like(out_ref)

        def hop(h, carry):
            g = lax.rem(my_pos - h + N_DEV, N_DEV)
            nxt = jnp.minimum(h + 1, N_DEV - 1)

            rdma = pltpu.make_async_remote_copy(
                src_ref=comm_ref.at[h],
                dst_ref=comm_ref.at[nxt],
                send_sem=send_sems.at[h],
                recv_sem=recv_sems.at[nxt],
                device_id=(right,),
                device_id_type=pl.DeviceIdType.MESH,
            )

            @pl.when(h < N_DEV - 1)
            def _():
                rdma.start()

            kcp = pltpu.make_async_copy(
                k_hbm.at[pl.ds(g * HQ, HQ)], kg_ref, kv_sems.at[0])
            vcp = pltpu.make_async_copy(
                v_hbm.at[pl.ds(g * HQ, HQ)], vg_ref, kv_sems.at[1])
            kcp.start()
            vcp.start()
            kcp.wait()
            vcp.wait()

            wq_g = comm_ref[h, 0]
            wo_g = comm_ref[h, 1]

            for c in range(N_CHUNK):
                klen = QC * (c + 1)
                xc = x_ref[c * QC:(c + 1) * QC, :]
                qb_i = (lax.broadcasted_iota(jnp.int32, (QC, klen), 0)
                        + c * QC) // BLK
                kb_i = lax.broadcasted_iota(jnp.int32, (QC, klen), 1) // BLK
                neg = jnp.where(kb_i <= qb_i, 0.0, -1e9)
                ctx_parts = []
                for hd in range(HQ):
                    qc = jnp.dot(
                        xc, wq_g[:, hd * DH:(hd + 1) * DH],
                        preferred_element_type=jnp.float32).astype(jnp.bfloat16)
                    s = lax.dot_general(
                        qc, kg_ref[hd, :klen],
                        (((1,), (1,)), ((), ())),
                        preferred_element_type=jnp.float32) * SCALE + neg
                    m = jnp.max(s, axis=1, keepdims=True)
                    p = jnp.exp(s - m)
                    p = p / jnp.sum(p, axis=1, keepdims=True)
                    ctx_parts.append(
                        jnp.dot(p.astype(jnp.bfloat16), vg_ref[hd, :klen],
                                preferred_element_type=jnp.float32
                                ).astype(jnp.bfloat16))
                ctx_c = jnp.concatenate(ctx_parts, axis=1)
                out_ref[c * QC:(c + 1) * QC, :] += jnp.dot(
                    ctx_c, wo_g, preferred_element_type=jnp.float32)

            @pl.when(h < N_DEV - 1)
            def _():
                rdma.wait()

            return carry

        lax.fori_loop(0, N_DEV, hop, 0)

        @functools.partial(pl.run_scoped,
                           sem2=pltpu.SemaphoreType.REGULAR)
        def _(sem2):
            for nbr in (left, right):
                pl.semaphore_signal(sem2, inc=1, device_id=(nbr,),
                                    device_id_type=pl.DeviceIdType.MESH)
            pl.semaphore_wait(sem2, 2)

    out = pl.pallas_call(
        body,
        out_shape=jax.ShapeDtypeStruct((SQ, DM), jnp.float32),
        in_specs=[
            pl.BlockSpec(memory_space=pltpu.VMEM),
            pl.BlockSpec(memory_space=pl.ANY),
            pl.BlockSpec(memory_space=pl.ANY),
            pl.BlockSpec(memory_space=pltpu.VMEM),
            pl.BlockSpec(memory_space=pltpu.VMEM),
        ],
        out_specs=pl.BlockSpec(memory_space=pltpu.VMEM),
        scratch_shapes=[
            pltpu.VMEM((N_DEV, 2, DM, DM), jnp.bfloat16),
            pltpu.VMEM((HQ, SKV, DH), jnp.bfloat16),
            pltpu.VMEM((HQ, SKV, DH), jnp.bfloat16),
            pltpu.SemaphoreType.DMA((N_DEV,)),
            pltpu.SemaphoreType.DMA((N_DEV,)),
            pltpu.SemaphoreType.DMA((2,)),
        ],
        compiler_params=pltpu.CompilerParams(
            collective_id=0,
            vmem_limit_bytes=56 * 1024 * 1024,
        ),
    )(xb, kb, vb, wq, wo)
    return out[None]


# device time: 282145 ns/iter; 1.2497x vs baseline; 1.2497x over previous
import functools

import jax
import jax.numpy as jnp
from jax import lax
from jax.experimental import pallas as pl
from jax.experimental.pallas import tpu as pltpu

N_DEV = 4
SQ = 2048
SKV = 2048
HQ = 8
DH = 128
DM = 1024
BLK = 64
QC = 512
N_CHUNK = SQ // QC
SCALE = 0.08838834764831843


def kernel(x, Wq, K_ext, V_ext, Wo):
    my = lax.axis_index("i")
    xb = x[0].astype(jnp.bfloat16)
    wq = (Wq * SCALE).astype(jnp.bfloat16)
    wo = Wo.astype(jnp.bfloat16)
    kb = lax.dynamic_index_in_dim(K_ext, my, 0, keepdims=False)
    vb = lax.dynamic_index_in_dim(V_ext, my, 0, keepdims=False)
    kb = jnp.transpose(kb, (1, 0, 2)).astype(jnp.bfloat16)
    vb = jnp.transpose(vb, (1, 0, 2)).astype(jnp.bfloat16)

    def body(x_ref, k_hbm, v_hbm, wq_ref, wo_ref, out_ref,
             comm_ref, kg_ref, vg_ref,
             send_sems, recv_sems, kv_sems):
        my_pos = lax.axis_index("i")
        left = lax.rem(my_pos + N_DEV - 1, N_DEV)
        right = lax.rem(my_pos + 1, N_DEV)

        barrier = pltpu.get_barrier_semaphore()
        for nbr in (left, right):
            pl.semaphore_signal(barrier, inc=1, device_id=(nbr,),
                                device_id_type=pl.DeviceIdType.MESH)
        pl.semaphore_wait(barrier, 2)

        comm_ref[0, 0, :, :] = wq_ref[:, :]
        comm_ref[0, 1, :, :] = wo_ref[:, :]
        out_ref[...] = jnp.zeros_like(out_ref)

        def hop(h, carry):
            g = lax.rem(my_pos - h + N_DEV, N_DEV)
            nxt = jnp.minimum(h + 1, N_DEV - 1)

            rdma = pltpu.make_async_remote_copy(
                src_ref=comm_ref.at[h],
                dst_ref=comm_ref.at[nxt],
                send_sem=send_sems.at[h],
                recv_sem=recv_sems.at[nxt],
                device_id=(right,),
                device_id_type=pl.DeviceIdType.MESH,
            )

            @pl.when(h < N_DEV - 1)
            def _():
                rdma.start()

            kcp = pltpu.make_async_copy(
                k_hbm.at[pl.ds(g * HQ, HQ)], kg_ref, kv_sems.at[0])
            vcp = pltpu.make_async_copy(
                v_hbm.at[pl.ds(g * HQ, HQ)], vg_ref, kv_sems.at[1])
            kcp.start()
            vcp.start()
            kcp.wait()
            vcp.wait()

            wq_g = comm_ref[h, 0]
            wo_g = comm_ref[h, 1]

            for c in range(N_CHUNK):
                klen = QC * (c + 1)
                xc = x_ref[c * QC:(c + 1) * QC, :]
                q_c = jnp.dot(
                    xc, wq_g,
                    preferred_element_type=jnp.float32).astype(jnp.bfloat16)
                qb_i = (lax.broadcasted_iota(jnp.int32, (QC, klen), 0)
                        + c * QC) // BLK
                kb_i = lax.broadcasted_iota(jnp.int32, (QC, klen), 1) // BLK
                neg = jnp.where(kb_i <= qb_i, 0.0, -1e9)
                ctx_parts = []
                for hd in range(HQ):
                    s = lax.dot_general(
                        q_c[:, hd * DH:(hd + 1) * DH], kg_ref[hd, :klen],
                        (((1,), (1,)), ((), ())),
                        preferred_element_type=jnp.float32) + neg
                    p = jnp.exp(s)
                    denom = jnp.sum(p, axis=1, keepdims=True)
                    ctx = jnp.dot(p.astype(jnp.bfloat16), vg_ref[hd, :klen],
                                  preferred_element_type=jnp.float32)
                    ctx_parts.append((ctx / denom).astype(jnp.bfloat16))
                ctx_c = jnp.concatenate(ctx_parts, axis=1)
                out_ref[c * QC:(c + 1) * QC, :] += jnp.dot(
                    ctx_c, wo_g, preferred_element_type=jnp.float32)

            @pl.when(h < N_DEV - 1)
            def _():
                rdma.wait()

            return carry

        lax.fori_loop(0, N_DEV, hop, 0)

        @functools.partial(pl.run_scoped,
                           sem2=pltpu.SemaphoreType.REGULAR)
        def _(sem2):
            for nbr in (left, right):
                pl.semaphore_signal(sem2, inc=1, device_id=(nbr,),
                                    device_id_type=pl.DeviceIdType.MESH)
            pl.semaphore_wait(sem2, 2)

    out = pl.pallas_call(
        body,
        out_shape=jax.ShapeDtypeStruct((SQ, DM), jnp.float32),
        in_specs=[
            pl.BlockSpec(memory_space=pltpu.VMEM),
            pl.BlockSpec(memory_space=pl.ANY),
            pl.BlockSpec(memory_space=pl.ANY),
            pl.BlockSpec(memory_space=pltpu.VMEM),
            pl.BlockSpec(memory_space=pltpu.VMEM),
        ],
        out_specs=pl.BlockSpec(memory_space=pltpu.VMEM),
        scratch_shapes=[
            pltpu.VMEM((N_DEV, 2, DM, DM), jnp.bfloat16),
            pltpu.VMEM((HQ, SKV, DH), jnp.bfloat16),
            pltpu.VMEM((HQ, SKV, DH), jnp.bfloat16),
            pltpu.SemaphoreType.DMA((N_DEV,)),
            pltpu.SemaphoreType.DMA((N_DEV,)),
            pltpu.SemaphoreType.DMA((2,)),
        ],
        compiler_params=pltpu.CompilerParams(
            collective_id=0,
            vmem_limit_bytes=56 * 1024 * 1024,
        ),
    )(xb, kb, vb, wq, wo)
    return out[None]
